# Initial kernel scaffold; baseline (speedup 1.0000x reference)
import jax
import jax.numpy as jnp
from jax.experimental import pallas as pl


def kernel(input, offset, weights):
    raise NotImplementedError("write your pallas kernel here")


# speedup vs baseline: 1.2732x; 1.2732x over previous
import functools

import jax
import jax.numpy as jnp
from jax import lax
from jax.experimental import pallas as pl
from jax.experimental.pallas import tpu as pltpu
from jax.experimental.pallas import tpu_sc as plsc


def _make_sc_gather(num_rows: int, dim: int, table_rows: int):
    info = plsc.get_sparse_core_info()
    nc, ns = info.num_cores, info.num_subcores
    nw = nc * ns
    assert num_rows % nw == 0
    rows_per_w = num_rows // nw
    chunk = 32
    assert rows_per_w % chunk == 0
    n_chunks = rows_per_w // chunk

    mesh = plsc.VectorSubcoreMesh(core_axis_name="c", subcore_axis_name="s")

    @functools.partial(
        pl.kernel,
        out_type=jax.ShapeDtypeStruct((num_rows, dim), jnp.float32),
        mesh=mesh,
        scratch_types=[
            pltpu.VMEM((rows_per_w,), jnp.int32),
            pltpu.VMEM((chunk, dim), jnp.float32),
            pltpu.VMEM((chunk, dim), jnp.float32),
            pltpu.SemaphoreType.DMA,
            pltpu.SemaphoreType.DMA,
            pltpu.SemaphoreType.DMA,
            pltpu.SemaphoreType.DMA,
        ],
    )
    def gather_kernel(table_hbm, idx_hbm, out_hbm, idx_v, buf0, buf1,
                      gsem0, gsem1, ssem0, ssem1):
        wid = lax.axis_index("s") * nc + lax.axis_index("c")
        base = wid * rows_per_w
        pltpu.sync_copy(idx_hbm.at[pl.ds(base, rows_per_w)], idx_v)

        bufs = (buf0, buf1)
        gsems = (gsem0, gsem1)
        ssems = (ssem0, ssem1)

        def gather_start(ch, slot):
            pltpu.make_async_copy(
                table_hbm.at[idx_v.at[pl.ds(ch * chunk, chunk)]],
                bufs[slot], gsems[slot]).start()

        def store_start(ch, slot):
            pltpu.make_async_copy(
                bufs[slot], out_hbm.at[pl.ds(base + ch * chunk, chunk)],
                ssems[slot]).start()

        gather_start(0, 0)
        for ch in range(n_chunks):
            slot = ch % 2
            nxt = 1 - slot
            pltpu.make_async_copy(
                table_hbm.at[idx_v.at[pl.ds(ch * chunk, chunk)]],
                bufs[slot], gsems[slot]).wait()
            if ch + 1 < n_chunks:
                if ch >= 1:
                    pltpu.make_async_copy(
                        bufs[nxt],
                        out_hbm.at[pl.ds(base + (ch - 1) * chunk, chunk)],
                        ssems[nxt]).wait()
                gather_start(ch + 1, nxt)
            store_start(ch, slot)
        if n_chunks >= 2:
            ch = n_chunks - 2
            pltpu.make_async_copy(
                bufs[ch % 2], out_hbm.at[pl.ds(base + ch * chunk, chunk)],
                ssems[ch % 2]).wait()
        ch = n_chunks - 1
        pltpu.make_async_copy(
            bufs[ch % 2], out_hbm.at[pl.ds(base + ch * chunk, chunk)],
            ssems[ch % 2]).wait()

    return gather_kernel


def kernel(input, offset, weights):
    seq_len = input.shape[0]
    table_rows, dim = weights.shape
    positions = jnp.clip(
        jnp.asarray(offset, jnp.int32) + jnp.arange(seq_len, dtype=jnp.int32),
        0, table_rows - 1)
    out = _make_sc_gather(seq_len, dim, table_rows)(weights, positions)
    return out[:, None, :]
